# baseline (speedup 1.0000x reference)
import jax
import jax.numpy as jnp
from jax.experimental import pallas as pl
from jax.experimental.pallas import tpu as pltpu


_BC = 1024


def _body(x_ref, t_ref, out_ref):
    x = x_ref[...]
    t = t_ref[...]
    masked = jnp.where(t > 0.0, x, -jnp.inf)
    m = jnp.max(masked, axis=0, keepdims=True)
    hp = jnp.max(t, axis=0, keepdims=True) > 0.0
    sig = jnp.clip(jax.nn.sigmoid(m), 1e-6, 1.0 - 1e-6)
    li = jnp.where(hp, -jnp.log(sig), 0.0)
    out_ref[...] = jnp.sum(li, axis=(0, 1), keepdims=True)[None]


@jax.jit
def kernel(input, target):
    B, C = input.shape
    xT = input.T
    tT = target.T
    nb = B // _BC
    parts = pl.pallas_call(
        _body,
        grid=(nb,),
        in_specs=[
            pl.BlockSpec((C, _BC), lambda i: (0, i)),
            pl.BlockSpec((C, _BC), lambda i: (0, i)),
        ],
        out_specs=pl.BlockSpec((1, 1, 1), lambda i: (i, 0, 0)),
        out_shape=jax.ShapeDtypeStruct((nb, 1, 1), jnp.float32),
        compiler_params=pltpu.CompilerParams(
            dimension_semantics=("parallel",),
        ),
    )(xT, tT)
    return jnp.sum(parts) / B
